# Initial kernel scaffold; baseline (speedup 1.0000x reference)
import jax
import jax.numpy as jnp
from jax.experimental import pallas as pl


def kernel(token_ids, weight_float):
    raise NotImplementedError("write your pallas kernel here")


# speedup vs baseline: 8.8796x; 8.8796x over previous
import functools
import math

import jax
import jax.numpy as jnp
from jax import lax
from jax.experimental import pallas as pl
from jax.experimental.pallas import tpu as pltpu
from jax.experimental.pallas import tpu_sc as plsc

_BITS = 32
_ROW_PAD = 128


def _build_sc_kernel(B, V, D):
    info = plsc.get_sparse_core_info()
    NC, NS, L = info.num_cores, info.num_subcores, info.num_lanes
    NW = NC * NS
    assert D == L, (D, L)
    assert B % NW == 0
    b_per_w = B // NW
    out_words = b_per_w * D * _BITS

    mesh = plsc.VectorSubcoreMesh(core_axis_name="c", subcore_axis_name="s")

    @functools.partial(
        pl.kernel,
        mesh=mesh,
        compiler_params=pltpu.CompilerParams(needs_layout_passes=False),
        out_type=jax.ShapeDtypeStruct((B * D * _BITS,), jnp.float32),
        scratch_types=[
            pltpu.VMEM((b_per_w,), jnp.int32),
            pltpu.VMEM((b_per_w, _ROW_PAD), jnp.float32),
            pltpu.VMEM((out_words,), jnp.float32),
            pltpu.SemaphoreType.DMA,
        ],
    )
    def sc_kernel(tok_hbm, w_hbm, out_hbm, idx_v, rows_v, out_v, sem):
        wid = lax.axis_index("s") * NC + lax.axis_index("c")
        base = wid * b_per_w
        pltpu.sync_copy(tok_hbm.at[pl.ds(base, b_per_w)], idx_v)
        pltpu.async_copy(w_hbm.at[idx_v], rows_v, sem).wait()

        lane = lax.iota(jnp.int32, L)
        lane32 = lane * _BITS

        def tok_body(i, carry):
            r = rows_v[i, pl.ds(0, L)]
            bits = lax.bitcast_convert_type(r, jnp.int32)
            off = i * (D * _BITS)
            for k in range(_BITS):
                bit = lax.shift_right_logical(bits, 31 - k) & 1
                f = bit.astype(jnp.float32)
                plsc.store_scatter(out_v, [lane32 + (off + k)], f)
            return carry

        lax.fori_loop(0, b_per_w, tok_body, 0)
        pltpu.sync_copy(out_v, out_hbm.at[pl.ds(base * D * _BITS, out_words)])

    return sc_kernel


def kernel(token_ids, weight_float):
    B = token_ids.shape[0]
    V, D = weight_float.shape
    tok = token_ids.astype(jnp.int32)
    w_pad = jnp.pad(weight_float, ((0, 0), (0, _ROW_PAD - D)))
    sc = _build_sc_kernel(B, V, D)
    out_flat = sc(tok, w_pad)
    return out_flat.reshape(B, D, _BITS)


# speedup vs baseline: 9.9372x; 1.1191x over previous
import functools
import math

import jax
import jax.numpy as jnp
from jax import lax
from jax.experimental import pallas as pl
from jax.experimental.pallas import tpu as pltpu
from jax.experimental.pallas import tpu_sc as plsc

_BITS = 32
_ROW_PAD = 128


def _build_sc_kernel(B, V, D):
    info = plsc.get_sparse_core_info()
    NC, NS, L = info.num_cores, info.num_subcores, info.num_lanes
    NW = NC * NS
    assert D == L, (D, L)
    assert B % NW == 0
    b_per_w = B // NW
    out_words = b_per_w * D * _BITS

    mesh = plsc.VectorSubcoreMesh(core_axis_name="c", subcore_axis_name="s")

    @functools.partial(
        pl.kernel,
        mesh=mesh,
        compiler_params=pltpu.CompilerParams(needs_layout_passes=False),
        out_type=jax.ShapeDtypeStruct((B, D, _BITS), jnp.float32),
        scratch_types=[
            pltpu.VMEM((b_per_w,), jnp.int32),
            pltpu.VMEM((b_per_w, _ROW_PAD), jnp.float32),
            pltpu.VMEM((b_per_w, D, _BITS), jnp.float32),
            pltpu.SemaphoreType.DMA,
        ],
    )
    def sc_kernel(tok_hbm, w_hbm, out_hbm, idx_v, rows_v, out_v, sem):
        wid = lax.axis_index("s") * NC + lax.axis_index("c")
        base = wid * b_per_w
        pltpu.sync_copy(tok_hbm.at[pl.ds(base, b_per_w)], idx_v)
        pltpu.async_copy(w_hbm.at[idx_v], rows_v, sem).wait()

        lane = lax.iota(jnp.int32, L)

        def tok_body(i, carry):
            r = rows_v[i, pl.ds(0, L)]
            bits = lax.bitcast_convert_type(r, jnp.int32)
            i_vec = jnp.full((L,), i, jnp.int32)
            for k in range(_BITS):
                bit = lax.shift_right_logical(bits, 31 - k) & 1
                f = bit.astype(jnp.float32)
                k_vec = jnp.full((L,), k, jnp.int32)
                plsc.store_scatter(out_v, [i_vec, lane, k_vec], f)
            return carry

        lax.fori_loop(0, b_per_w, tok_body, 0)
        pltpu.sync_copy(out_v, out_hbm.at[pl.ds(base, b_per_w)])

    return sc_kernel


def kernel(token_ids, weight_float):
    B = token_ids.shape[0]
    V, D = weight_float.shape
    tok = token_ids.astype(jnp.int32)
    w_pad = jnp.pad(weight_float, ((0, 0), (0, _ROW_PAD - D)))
    sc = _build_sc_kernel(B, V, D)
    return sc(tok, w_pad)


# speedup vs baseline: 12.0922x; 1.2169x over previous
import functools
import math

import jax
import jax.numpy as jnp
from jax import lax
from jax.experimental import pallas as pl
from jax.experimental.pallas import tpu as pltpu
from jax.experimental.pallas import tpu_sc as plsc

_BITS = 32
_ROW_PAD = 128


def _build_sc_kernel(B, V, D):
    info = plsc.get_sparse_core_info()
    NC, NS, L = info.num_cores, info.num_subcores, info.num_lanes
    NW = NC * NS
    assert D == L, (D, L)
    assert B % NW == 0
    b_per_w = B // NW
    out_words = b_per_w * D * _BITS

    mesh = plsc.VectorSubcoreMesh(core_axis_name="c", subcore_axis_name="s")

    @functools.partial(
        pl.kernel,
        mesh=mesh,
        compiler_params=pltpu.CompilerParams(needs_layout_passes=False),
        out_type=jax.ShapeDtypeStruct((B, D, _BITS), jnp.float32),
        scratch_types=[
            pltpu.VMEM((b_per_w,), jnp.int32),
            pltpu.VMEM((b_per_w, _ROW_PAD), jnp.float32),
            pltpu.VMEM((b_per_w, D, _BITS), jnp.float32),
            pltpu.SemaphoreType.DMA,
        ],
    )
    def sc_kernel(tok_hbm, w_hbm, out_hbm, idx_v, rows_v, out_v, sem):
        wid = lax.axis_index("s") * NC + lax.axis_index("c")
        base = wid * b_per_w
        pltpu.sync_copy(tok_hbm.at[pl.ds(base, b_per_w)], idx_v)
        pltpu.async_copy(w_hbm.at[idx_v], rows_v, sem).wait()

        lane = lax.iota(jnp.int32, L)
        shamt_hi = 31 - lane
        shamt_lo = 15 - lane
        dnums = lax.GatherDimensionNumbers(
            offset_dims=(), collapsed_slice_dims=(0,), start_index_map=(0,)
        )

        def tok_body(i, carry):
            r = rows_v[i, pl.ds(0, L)]
            bits = lax.bitcast_convert_type(r, jnp.int32)
            for d in range(D):
                bc = lax.gather(
                    bits,
                    jnp.full((L, 1), d, jnp.int32),
                    dnums,
                    (1,),
                    mode=lax.GatherScatterMode.PROMISE_IN_BOUNDS,
                )
                hi = (lax.shift_right_logical(bc, shamt_hi) & 1).astype(jnp.float32)
                lo = (lax.shift_right_logical(bc, shamt_lo) & 1).astype(jnp.float32)
                out_v[i, d, pl.ds(0, L)] = hi
                out_v[i, d, pl.ds(L, L)] = lo
            return carry

        lax.fori_loop(0, b_per_w, tok_body, 0)
        pltpu.sync_copy(out_v, out_hbm.at[pl.ds(base, b_per_w)])

    return sc_kernel


def kernel(token_ids, weight_float):
    B = token_ids.shape[0]
    V, D = weight_float.shape
    tok = token_ids.astype(jnp.int32)
    w_pad = jnp.pad(weight_float, ((0, 0), (0, _ROW_PAD - D)))
    sc = _build_sc_kernel(B, V, D)
    return sc(tok, w_pad)


# speedup vs baseline: 15.8847x; 1.3136x over previous
import functools

import jax
import jax.numpy as jnp
from jax import lax
from jax.experimental import pallas as pl
from jax.experimental.pallas import tpu as pltpu
from jax.experimental.pallas import tpu_sc as plsc

_BITS = 32


def _build_sc_kernel(B, V, D, Vpad):
    info = plsc.get_sparse_core_info()
    NC, NS, L = info.num_cores, info.num_subcores, info.num_lanes
    NW = NC * NS
    assert D * _BITS == L * NW
    n_groups = B // L
    half = _BITS // 2

    mesh = plsc.VectorSubcoreMesh(core_axis_name="c", subcore_axis_name="s")

    @functools.partial(
        pl.kernel,
        mesh=mesh,
        compiler_params=pltpu.CompilerParams(needs_layout_passes=False),
        out_type=jax.ShapeDtypeStruct((D * _BITS, B), jnp.float32),
        scratch_types=[
            pltpu.VMEM((B,), jnp.int32),
            pltpu.VMEM((Vpad,), jnp.float32),
            pltpu.VMEM((L, B), jnp.float32),
            pltpu.SemaphoreType.DMA,
        ],
    )
    def sc_kernel(tok_hbm, wt_hbm, out_hbm, tok_v, trow_v, out_v, sem):
        wid = lax.axis_index("s") * NC + lax.axis_index("c")
        d = lax.shift_right_logical(wid, 1)
        kh = wid & 1
        pltpu.sync_copy(tok_hbm, tok_v)
        pltpu.sync_copy(wt_hbm.at[d], trow_v)

        sh_base = 31 - kh * half

        def grp_body(gr, carry):
            t = tok_v[pl.ds(gr * L, L)]
            vals = plsc.load_gather(trow_v, [t])
            bits = lax.bitcast_convert_type(vals, jnp.int32)
            for j in range(half):
                bit = lax.shift_right_logical(bits, sh_base - j) & 1
                out_v[j, pl.ds(gr * L, L)] = bit.astype(jnp.float32)
            return carry

        lax.fori_loop(0, n_groups, grp_body, 0)

        pltpu.sync_copy(out_v, out_hbm.at[pl.ds(wid * L, L)])

    return sc_kernel


def kernel(token_ids, weight_float):
    B = token_ids.shape[0]
    V, D = weight_float.shape
    Vpad = 1024 if V <= 1024 else -(-V // 128) * 128
    tok = token_ids.astype(jnp.int32)
    w_t = jnp.pad(weight_float.T, ((0, 0), (0, Vpad - V)))
    sc = _build_sc_kernel(B, V, D, Vpad)
    out2 = sc(tok, w_t)
    return jnp.transpose(out2.reshape(D, _BITS, B), (2, 0, 1))
